# baseline (speedup 1.0000x reference)
import jax
import jax.numpy as jnp
from jax.experimental import pallas as pl


_BM = 512


def _support_kernel(x_ref, w_ref, o_ref):
    o_ref[...] = jnp.dot(x_ref[...], w_ref[...],
                         preferred_element_type=jnp.float32)


def _spmm_kernel(adj_ref, s_ref, b_ref, o_ref):
    acc = jnp.dot(adj_ref[...], s_ref[...], preferred_element_type=jnp.float32)
    o_ref[...] = jnp.maximum(acc + b_ref[...], 0.0)


def _layer(x, adj, W, b):
    n, d_in = x.shape
    d_out = W.shape[1]
    support = pl.pallas_call(
        _support_kernel,
        grid=(n // _BM,),
        in_specs=[
            pl.BlockSpec((_BM, d_in), lambda i: (i, 0)),
            pl.BlockSpec((d_in, d_out), lambda i: (0, 0)),
        ],
        out_specs=pl.BlockSpec((_BM, d_out), lambda i: (i, 0)),
        out_shape=jax.ShapeDtypeStruct((n, d_out), jnp.float32),
    )(x, W)
    out = pl.pallas_call(
        _spmm_kernel,
        grid=(n // _BM,),
        in_specs=[
            pl.BlockSpec((_BM, n), lambda i: (i, 0)),
            pl.BlockSpec((n, d_out), lambda i: (0, 0)),
            pl.BlockSpec((1, d_out), lambda i: (0, 0)),
        ],
        out_specs=pl.BlockSpec((_BM, d_out), lambda i: (i, 0)),
        out_shape=jax.ShapeDtypeStruct((n, d_out), jnp.float32),
    )(adj, support, b.reshape(1, d_out))
    return out


def kernel(x, adj, W1, b1, W2, b2, W3, b3):
    h = _layer(x, adj, W1, b1)
    h = _layer(h, adj, W2, b2)
    return _layer(h, adj, W3, b3)
